# baseline (speedup 1.0000x reference)
import functools

import jax
import jax.numpy as jnp
from jax import lax
from jax.experimental import pallas as pl
from jax.experimental.pallas import tpu as pltpu
from jax.experimental.pallas import tpu_sc as plsc

_VOCAB = 1000000
_D = 64
_B = 16384
_NNEG = 5
_NW = 32
_G = 128
_BW = _B // _NW
_NZW = _B * _NNEG // _NW
_RW = 2 * _BW + _NZW
_NG = _RW // _G
_GP = _G // 8

_mesh = plsc.VectorSubcoreMesh(core_axis_name="c", subcore_axis_name="s")


@functools.partial(
    pl.kernel,
    mesh=_mesh,
    out_type=(
        jax.ShapeDtypeStruct((_B, _D), jnp.float32),
        jax.ShapeDtypeStruct((_B, _D), jnp.float32),
        jax.ShapeDtypeStruct((_B * _NNEG, _D), jnp.float32),
    ),
    scratch_types=[
        pltpu.VMEM((_RW,), jnp.int32),
        pltpu.VMEM((_G, 2 * _D), jnp.float32),
        pltpu.VMEM((_G, 2 * _D), jnp.float32),
        pltpu.VMEM((_G, 2 * _D), jnp.float32),
        pltpu.VMEM((_G, _D), jnp.float32),
        pltpu.VMEM((_G, _D), jnp.float32),
        pltpu.VMEM((_GP, 128), jnp.int32),
        pltpu.VMEM((_GP, 128), jnp.int32),
        pltpu.SemaphoreType.DMA,
        pltpu.SemaphoreType.DMA,
        pltpu.SemaphoreType.DMA,
        pltpu.SemaphoreType.DMA,
    ],
)
def _sgns(in_emb, out_emb, u_all, p_rep,
          o_center, o_context, o_noise,
          idx_u, pair0, pair1, pair2, comp0, comp1, par0, par1,
          sem_g, sem_p, sem_o0, sem_o1):
    wid = lax.axis_index("s") * 2 + lax.axis_index("c")
    pairs = (pair0, pair1, pair2)
    comps = (comp0, comp1)
    pars = (par0, par1)
    sems_o = (sem_o0, sem_o1)

    pltpu.sync_copy(u_all.at[pl.ds(wid * _RW, _RW)], idx_u)

    tasks = (
        [(in_emb, o_center, wid * _BW + k * _G) for k in range(_BW // _G)]
        + [(out_emb, o_context, wid * _BW + k * _G) for k in range(_BW // _G)]
        + [(out_emb, o_noise, wid * _NZW + k * _G) for k in range(_NZW // _G)]
    )

    def fire_par(g):
        return pltpu.async_copy(
            p_rep.at[pl.ds(wid * (_RW // 8) + g * _GP, _GP)], pars[g % 2], sem_p)

    def fire_gather(g):
        table = tasks[g][0]
        return pltpu.async_copy(
            table.at[idx_u.at[pl.ds(g * _G, _G)]], pairs[g % 3], sem_g)

    def compact(g):
        src = pairs[g % 3]
        dst = comps[g % 2]
        par = pars[g % 2]

        def body(j, _):
            m = par[j >> 3, pl.ds((j & 7) * 16, 16)] != 0
            for k in range(_D // 16):
                left = src[j, pl.ds(k * 16, 16)]
                right = src[j, pl.ds(_D + k * 16, 16)]
                dst[j, pl.ds(k * 16, 16)] = jnp.where(m, right, left)
            return 0

        lax.fori_loop(0, _G, body, 0)

    def fire_out(g):
        _, out, base = tasks[g]
        return pltpu.async_copy(
            comps[g % 2], out.at[pl.ds(base, _G)], sems_o[g % 2])

    gh = [None] * _NG
    ph = [None] * _NG
    oh = [None] * _NG
    ph[0] = fire_par(0)
    ph[1] = fire_par(1)
    for g in range(_NG):
        gh[g] = fire_gather(g)
        if g >= 1:
            c = g - 1
            gh[c].wait()
            ph[c].wait()
            if c >= 2:
                oh[c - 2].wait()
            compact(c)
            oh[c] = fire_out(c)
            if g + 1 < _NG:
                ph[g + 1] = fire_par(g + 1)
    c = _NG - 1
    gh[c].wait()
    ph[c].wait()
    oh[c - 2].wait()
    compact(c)
    oh[c] = fire_out(c)
    oh[_NG - 2].wait()
    oh[_NG - 1].wait()


def kernel(input_embeddings, output_embeddings, center_words, context_words, noise_words):
    v_w = jnp.concatenate([
        center_words.astype(jnp.int32).reshape(_NW, _BW),
        context_words.astype(jnp.int32).reshape(_NW, _BW),
        noise_words.astype(jnp.int32).reshape(_NW, _NZW),
    ], axis=1).reshape(_NW * _RW)
    u_all = v_w >> 1
    p_rep = jnp.broadcast_to(
        (v_w & 1)[:, None], (_NW * _RW, 16)).reshape(_NW * _RW // 8, 128)
    o_center, o_context, o_noise = _sgns(
        input_embeddings.reshape(_VOCAB // 2, 2 * _D),
        output_embeddings.reshape(_VOCAB // 2, 2 * _D),
        u_all, p_rep)
    return (o_center, o_context, o_noise.reshape(_B, _NNEG, _D))


# speedup vs baseline: 1.0441x; 1.0441x over previous
import functools

import jax
import jax.numpy as jnp
from jax import lax
from jax.experimental import pallas as pl
from jax.experimental.pallas import tpu as pltpu
from jax.experimental.pallas import tpu_sc as plsc

_VOCAB = 1000000
_D = 64
_B = 16384
_NNEG = 5
_NW = 32
_G = 512
_BW = _B // _NW
_NZW = _B * _NNEG // _NW
_NG = (2 * _BW + _NZW) // _G
_NSLOT = 3

_mesh = plsc.VectorSubcoreMesh(core_axis_name="c", subcore_axis_name="s")


@functools.partial(
    pl.kernel,
    mesh=_mesh,
    compiler_params=pltpu.CompilerParams(use_tc_tiling_on_sc=False),
    out_type=(
        jax.ShapeDtypeStruct((_B, _D), jnp.float32),
        jax.ShapeDtypeStruct((_B, _D), jnp.float32),
        jax.ShapeDtypeStruct((_B * _NNEG, _D), jnp.float32),
    ),
    scratch_types=[
        pltpu.VMEM((_NG * _G,), jnp.int32),
        pltpu.VMEM((_G, _D), jnp.float32),
        pltpu.VMEM((_G, _D), jnp.float32),
        pltpu.VMEM((_G, _D), jnp.float32),
        pltpu.SemaphoreType.DMA,
        pltpu.SemaphoreType.DMA,
        pltpu.SemaphoreType.DMA,
        pltpu.SemaphoreType.DMA,
    ],
)
def _sgns(in_emb, out_emb, center, context, noise,
          o_center, o_context, o_noise,
          idx, buf0, buf1, buf2, sem_g, sem_o0, sem_o1, sem_o2):
    wid = lax.axis_index("s") * 2 + lax.axis_index("c")
    bufs = (buf0, buf1, buf2)
    sems_o = (sem_o0, sem_o1, sem_o2)

    pltpu.sync_copy(center.at[pl.ds(wid * _BW, _BW)], idx.at[pl.ds(0, _BW)])
    pltpu.sync_copy(context.at[pl.ds(wid * _BW, _BW)], idx.at[pl.ds(_BW, _BW)])
    pltpu.sync_copy(noise.at[pl.ds(wid * _NZW, _NZW)], idx.at[pl.ds(2 * _BW, _NZW)])

    tasks = (
        [(in_emb, o_center, wid * _BW)]
        + [(out_emb, o_context, wid * _BW)]
        + [(out_emb, o_noise, wid * _NZW + k * _G) for k in range(_NZW // _G)]
    )

    def fire_gather(g):
        table = tasks[g][0]
        return pltpu.async_copy(
            table.at[idx.at[pl.ds(g * _G, _G)]], bufs[g % _NSLOT], sem_g)

    def fire_out(g):
        _, out, base = tasks[g]
        return pltpu.async_copy(
            bufs[g % _NSLOT], out.at[pl.ds(base, _G)], sems_o[g % _NSLOT])

    gh = [None] * _NG
    oh = [None] * _NG
    for g in range(_NG):
        if g >= _NSLOT:
            oh[g - _NSLOT].wait()
        gh[g] = fire_gather(g)
        if g >= 1:
            gh[g - 1].wait()
            oh[g - 1] = fire_out(g - 1)
    gh[_NG - 1].wait()
    oh[_NG - 1] = fire_out(_NG - 1)
    for g in range(_NG - _NSLOT, _NG):
        oh[g].wait()


def kernel(input_embeddings, output_embeddings, center_words, context_words, noise_words):
    center1d = center_words.astype(jnp.int32)
    context1d = context_words.astype(jnp.int32)
    noise1d = noise_words.astype(jnp.int32).reshape(_B * _NNEG)
    o_center, o_context, o_noise = _sgns(
        input_embeddings, output_embeddings, center1d, context1d, noise1d)
    return (o_center, o_context, o_noise.reshape(_B, _NNEG, _D))
